# baseline (speedup 1.0000x reference)
import functools

import jax
import jax.numpy as jnp
from jax import lax
from jax.experimental import pallas as pl
from jax.experimental.pallas import tpu as pltpu
from jax.experimental.pallas import tpu_sc as plsc

B = 1024
K = 65536
D = 128
C = 1000
CPAD = 1024
KBLK = 4096
NBLK = K // KBLK

NC = 2
NS = 16
NW = NC * NS
RPW = K // NW
CHUNK = 128
NCH = RPW // CHUNK


def _sc_segment_kernel(q_hbm, lab_hbm, sums_out, cnt_out,
                       idx_v, rows_v, ones_v, zero_v,
                       sh_sums, sh_cnt):
    cid = lax.axis_index("c")
    sid = lax.axis_index("s")
    wid = cid * NS + sid

    for r in range(CHUNK):
        for g in range(D // 16):
            ones_v[r, pl.ds(g * 16, 16)] = jnp.full((16,), 1.0, jnp.float32)

    @pl.when(sid == 0)
    def _():
        for r in range(CHUNK):
            for g in range(D // 16):
                zero_v[r, pl.ds(g * 16, 16)] = jnp.zeros((16,), jnp.float32)
        for j in range(CPAD // CHUNK):
            pltpu.sync_copy(zero_v, sh_sums.at[pl.ds(j * CHUNK, CHUNK)])
            pltpu.sync_copy(zero_v, sh_cnt.at[pl.ds(j * CHUNK, CHUNK)])

    plsc.subcore_barrier()

    def _chunk(ch, carry):
        base = wid * RPW + ch * CHUNK
        pltpu.sync_copy(lab_hbm.at[pl.ds(base, CHUNK)], idx_v)
        pltpu.sync_copy(q_hbm.at[pl.ds(base, CHUNK)], rows_v)
        pltpu.sync_copy(rows_v, sh_sums.at[idx_v], add=True)
        pltpu.sync_copy(ones_v, sh_cnt.at[idx_v], add=True)
        return carry

    lax.fori_loop(0, NCH, _chunk, 0)

    plsc.subcore_barrier()

    @pl.when(sid == 0)
    def _():
        pltpu.sync_copy(sh_sums, sums_out.at[cid])
        pltpu.sync_copy(sh_cnt, cnt_out.at[cid])


def _lanegroup_sum(x, width=128):
    parts = [x[:, g * width:(g + 1) * width] for g in range(x.shape[1] // width)]
    while len(parts) > 1:
        nxt = [parts[i] + parts[i + 1] for i in range(0, len(parts) - 1, 2)]
        if len(parts) % 2:
            nxt.append(parts[-1])
        parts = nxt
    return parts[0]


def _tc_kernel(sums2_ref, cnt2_ref, labels_ref, q_ref, bfm2_ref, out_ref,
               p_s, cnt_s, macc, tacc):
    i = pl.program_id(0)

    @pl.when(i == 0)
    def _():
        sums = sums2_ref[0] + sums2_ref[1]
        s2 = jnp.sum(sums * sums, axis=1, keepdims=True)
        centn = (sums * jax.lax.rsqrt(jnp.maximum(s2, 1e-24))).astype(
            jnp.bfloat16)
        bf = bfm2_ref[...] * jnp.bfloat16(-0.5)
        sim = jax.lax.dot_general(
            bf, centn, (((1,), (1,)), ((), ())),
            preferred_element_type=jnp.float32)
        col = jax.lax.broadcasted_iota(jnp.int32, (B, CPAD), 1)
        sim = jnp.where(col < C, sim, -3.0)
        p = jnp.argmax(sim, axis=1).astype(jnp.int32)
        p_s[...] = p[:, None]
        onehot_p = (p[:, None] == col).astype(jnp.bfloat16)
        cnt16 = (cnt2_ref[0] + cnt2_ref[1]).astype(jnp.bfloat16)
        cntm = jax.lax.dot_general(
            onehot_p, cnt16, (((1,), (0,)), ((), ())),
            preferred_element_type=jnp.float32)
        cnt_s[...] = cntm[:, 0:1]
        macc[...] = jnp.zeros_like(macc)
        tacc[...] = jnp.zeros_like(tacc)

    @pl.when(i > 0)
    def _():
        simn2 = jax.lax.dot_general(
            bfm2_ref[...], q_ref[...], (((1,), (1,)), ((), ())),
            preferred_element_type=jnp.float32)
        t = simn2 + jnp.float32(2.0 + 1e-6)
        mae = (t * jax.lax.rsqrt(t)).astype(jnp.bfloat16)
        lab = labels_ref[0, 0, :]
        p16 = p_s[...].astype(jnp.int16)
        mask = p16 == lab[None, :]
        macc[...] += _lanegroup_sum(
            jnp.where(mask, mae, jnp.bfloat16(0.0))).astype(jnp.float32)
        tacc[...] += _lanegroup_sum(mae).astype(jnp.float32)

        @pl.when(i == NBLK)
        def _():
            m = jnp.sum(macc[...], axis=1)
            t_ = jnp.sum(tacc[...], axis=1)
            cnt = cnt_s[:, 0]
            minent = jnp.mean(m / (cnt + 1e-6))
            inter = jnp.mean((t_ - m) / ((jnp.float32(K) - cnt) + 1e-6))
            out_ref[...] = jnp.broadcast_to(minent + 2.0 - inter, (1, 1))


def kernel(batch_feature, queue_emb_copy, info_label):
    lab32 = info_label.astype(jnp.int32)
    labels3 = info_label.astype(jnp.int16).reshape(K // KBLK, 1, KBLK)
    q16 = queue_emb_copy.astype(jnp.bfloat16)
    bfm2 = (batch_feature * -2.0).astype(jnp.bfloat16)

    sc_seg = pl.kernel(
        _sc_segment_kernel,
        out_type=[
            jax.ShapeDtypeStruct((NC, CPAD, D), jnp.float32),
            jax.ShapeDtypeStruct((NC, CPAD, D), jnp.float32),
        ],
        scratch_types=[
            pltpu.VMEM((CHUNK,), jnp.int32),
            pltpu.VMEM((CHUNK, D), jnp.float32),
            pltpu.VMEM((CHUNK, D), jnp.float32),
            pltpu.VMEM((CHUNK, D), jnp.float32),
            pltpu.VMEM_SHARED((CPAD, D), jnp.float32),
            pltpu.VMEM_SHARED((CPAD, D), jnp.float32),
        ],
        mesh=plsc.VectorSubcoreMesh(core_axis_name="c", subcore_axis_name="s",
                                    num_cores=NC, num_subcores=NS),
    )
    sums2, cnt2 = sc_seg(queue_emb_copy, lab32)

    out = pl.pallas_call(
        _tc_kernel,
        grid=(NBLK + 1,),
        in_specs=[
            pl.BlockSpec((NC, CPAD, D), lambda i: (0, 0, 0)),
            pl.BlockSpec((NC, CPAD, D), lambda i: (0, 0, 0)),
            pl.BlockSpec((1, 1, KBLK), lambda i: (jnp.maximum(i - 1, 0), 0, 0)),
            pl.BlockSpec((KBLK, D), lambda i: (jnp.maximum(i - 1, 0), 0)),
            pl.BlockSpec((B, D), lambda i: (0, 0)),
        ],
        out_specs=pl.BlockSpec((1, 1), lambda i: (0, 0)),
        out_shape=jax.ShapeDtypeStruct((1, 1), jnp.float32),
        scratch_shapes=[
            pltpu.VMEM((B, 1), jnp.int32),
            pltpu.VMEM((B, 1), jnp.float32),
            pltpu.VMEM((B, 128), jnp.float32),
            pltpu.VMEM((B, 128), jnp.float32),
        ],
    )(sums2, cnt2, labels3, q16, bfm2)

    return out[0, 0]


# speedup vs baseline: 1.1387x; 1.1387x over previous
import functools

import jax
import jax.numpy as jnp
from jax import lax
from jax.experimental import pallas as pl
from jax.experimental.pallas import tpu as pltpu
from jax.experimental.pallas import tpu_sc as plsc

B = 1024
K = 65536
D = 128
C = 1000
CPAD = 1024
KBLK = 4096
NBLK = K // KBLK

NC = 2
NS = 16
NW = NC * NS
RPW = K // NW
CHUNK = 128
NCH = RPW // CHUNK


STAGE = 512
NSTG = RPW // STAGE
SCH = STAGE // CHUNK


def _sc_segment_kernel(q_hbm, lab2_hbm, sums_out,
                       lab_v, rows_v, zero_v, sh_sums):
    cid = lax.axis_index("c")
    sid = lax.axis_index("s")
    wid = cid * NS + sid

    @pl.when(sid == 0)
    def _():
        for r in range(CHUNK):
            for g in range(D // 16):
                zero_v[r, pl.ds(g * 16, 16)] = jnp.zeros((16,), jnp.float32)
        for j in range(CPAD // CHUNK):
            pltpu.sync_copy(zero_v, sh_sums.at[pl.ds(j * CHUNK, CHUNK)])

    pltpu.sync_copy(lab2_hbm.at[pl.ds(wid * NCH, NCH)], lab_v)

    plsc.subcore_barrier()

    def _stage(st, carry):
        base = wid * RPW + st * STAGE
        pltpu.sync_copy(q_hbm.at[pl.ds(base, STAGE)], rows_v)
        for c in range(SCH):
            pltpu.sync_copy(rows_v.at[pl.ds(c * CHUNK, CHUNK)],
                            sh_sums.at[lab_v.at[st * SCH + c]], add=True)
        return carry

    lax.fori_loop(0, NSTG, _stage, 0)

    plsc.subcore_barrier()

    @pl.when(sid == 0)
    def _():
        pltpu.sync_copy(sh_sums, sums_out.at[cid])


def _lanegroup_sum(x, width=128):
    parts = [x[:, g * width:(g + 1) * width] for g in range(x.shape[1] // width)]
    while len(parts) > 1:
        nxt = [parts[i] + parts[i + 1] for i in range(0, len(parts) - 1, 2)]
        if len(parts) % 2:
            nxt.append(parts[-1])
        parts = nxt
    return parts[0]


def _tc_kernel(sums2_ref, labels_ref, q_ref, bfm2_ref, out_ref,
               p_s, macc, tacc, cacc):
    i = pl.program_id(0)

    @pl.when(i == 0)
    def _():
        sums = sums2_ref[0] + sums2_ref[1]
        s2 = jnp.sum(sums * sums, axis=1, keepdims=True)
        centn = (sums * jax.lax.rsqrt(jnp.maximum(s2, 1e-24))).astype(
            jnp.bfloat16)
        bf = bfm2_ref[...] * jnp.bfloat16(-0.5)
        sim = jax.lax.dot_general(
            bf, centn, (((1,), (1,)), ((), ())),
            preferred_element_type=jnp.float32)
        col = jax.lax.broadcasted_iota(jnp.int32, (B, CPAD), 1)
        sim = jnp.where(col < C, sim, -3.0)
        p = jnp.argmax(sim, axis=1).astype(jnp.int32)
        p_s[...] = p[:, None]
        macc[...] = jnp.zeros_like(macc)
        tacc[...] = jnp.zeros_like(tacc)
        cacc[...] = jnp.zeros_like(cacc)

    @pl.when(i > 0)
    def _():
        simn2 = jax.lax.dot_general(
            bfm2_ref[...], q_ref[...], (((1,), (1,)), ((), ())),
            preferred_element_type=jnp.float32)
        t = simn2 + jnp.float32(2.0 + 1e-6)
        mae = (t * jax.lax.rsqrt(t)).astype(jnp.bfloat16)
        lab = labels_ref[0, 0, :]
        p16 = p_s[...].astype(jnp.int16)
        mask = p16 == lab[None, :]
        macc[...] += _lanegroup_sum(
            jnp.where(mask, mae, jnp.bfloat16(0.0))).astype(jnp.float32)
        tacc[...] += _lanegroup_sum(mae).astype(jnp.float32)
        cacc[...] += _lanegroup_sum(
            jnp.where(mask, jnp.bfloat16(1.0), jnp.bfloat16(0.0))
        ).astype(jnp.float32)

        @pl.when(i == NBLK)
        def _():
            m = jnp.sum(macc[...], axis=1)
            t_ = jnp.sum(tacc[...], axis=1)
            cnt = jnp.sum(cacc[...], axis=1)
            minent = jnp.mean(m / (cnt + 1e-6))
            inter = jnp.mean((t_ - m) / ((jnp.float32(K) - cnt) + 1e-6))
            out_ref[...] = jnp.broadcast_to(minent + 2.0 - inter, (1, 1))


def kernel(batch_feature, queue_emb_copy, info_label):
    lab2 = info_label.astype(jnp.int32).reshape(NW * NCH, CHUNK)
    labels3 = info_label.astype(jnp.int16).reshape(K // KBLK, 1, KBLK)
    q16 = queue_emb_copy.astype(jnp.bfloat16)
    bfm2 = (batch_feature * -2.0).astype(jnp.bfloat16)

    sc_seg = pl.kernel(
        _sc_segment_kernel,
        out_type=jax.ShapeDtypeStruct((NC, CPAD, D), jnp.float32),
        scratch_types=[
            pltpu.VMEM((NCH, CHUNK), jnp.int32),
            pltpu.VMEM((STAGE, D), jnp.float32),
            pltpu.VMEM((CHUNK, D), jnp.float32),
            pltpu.VMEM_SHARED((CPAD, D), jnp.float32),
        ],
        mesh=plsc.VectorSubcoreMesh(core_axis_name="c", subcore_axis_name="s",
                                    num_cores=NC, num_subcores=NS),
    )
    sums2 = sc_seg(queue_emb_copy, lab2)

    out = pl.pallas_call(
        _tc_kernel,
        grid=(NBLK + 1,),
        in_specs=[
            pl.BlockSpec((NC, CPAD, D), lambda i: (0, 0, 0)),
            pl.BlockSpec((1, 1, KBLK), lambda i: (jnp.maximum(i - 1, 0), 0, 0)),
            pl.BlockSpec((KBLK, D), lambda i: (jnp.maximum(i - 1, 0), 0)),
            pl.BlockSpec((B, D), lambda i: (0, 0)),
        ],
        out_specs=pl.BlockSpec((1, 1), lambda i: (0, 0)),
        out_shape=jax.ShapeDtypeStruct((1, 1), jnp.float32),
        scratch_shapes=[
            pltpu.VMEM((B, 1), jnp.int32),
            pltpu.VMEM((B, 128), jnp.float32),
            pltpu.VMEM((B, 128), jnp.float32),
            pltpu.VMEM((B, 128), jnp.float32),
        ],
    )(sums2, labels3, q16, bfm2)

    return out[0, 0]


# speedup vs baseline: 1.2072x; 1.0602x over previous
import functools

import jax
import jax.numpy as jnp
from jax import lax
from jax.experimental import pallas as pl
from jax.experimental.pallas import tpu as pltpu
from jax.experimental.pallas import tpu_sc as plsc

B = 1024
K = 65536
D = 128
C = 1000
CPAD = 1024
KBLK = 4096
NBLK = K // KBLK

NC = 2
NS = 16
NW = NC * NS
RPW = K // NW
CHUNK = 128
NCH = RPW // CHUNK


STAGE = 256
NSTG = RPW // STAGE
SCH = STAGE // CHUNK
ZROWS = CPAD // NS


def _sc_segment_kernel(q_hbm, lab2_hbm, sums_out,
                       lab_v, rows_a, rows_b, zero_v, sh_sums,
                       sem_a, sem_b):
    cid = lax.axis_index("c")
    sid = lax.axis_index("s")
    wid = cid * NS + sid

    for r in range(ZROWS):
        for g in range(D // 16):
            zero_v[r, pl.ds(g * 16, 16)] = jnp.zeros((16,), jnp.float32)
    pltpu.sync_copy(lab2_hbm.at[pl.ds(wid * NCH, NCH)], lab_v)
    pltpu.sync_copy(zero_v, sh_sums.at[pl.ds(sid * ZROWS, ZROWS)])

    plsc.subcore_barrier()

    bufs = (rows_a, rows_b)
    sems = (sem_a, sem_b)

    def _start(st):
        base = wid * RPW + st * STAGE
        pltpu.async_copy(q_hbm.at[pl.ds(base, STAGE)], bufs[st % 2],
                         sems[st % 2])

    def _wait(st):
        base = wid * RPW + st * STAGE
        pltpu.make_async_copy(q_hbm.at[pl.ds(base, STAGE)], bufs[st % 2],
                              sems[st % 2]).wait()

    _start(0)
    for st in range(NSTG):
        if st + 1 < NSTG:
            _start(st + 1)
        _wait(st)
        for c in range(SCH):
            pltpu.sync_copy(bufs[st % 2].at[pl.ds(c * CHUNK, CHUNK)],
                            sh_sums.at[lab_v.at[st * SCH + c]], add=True)

    plsc.subcore_barrier()

    @pl.when(sid == 0)
    def _():
        pltpu.sync_copy(sh_sums, sums_out.at[cid])


def _lanegroup_sum(x, width=128):
    parts = [x[:, g * width:(g + 1) * width] for g in range(x.shape[1] // width)]
    while len(parts) > 1:
        nxt = [parts[i] + parts[i + 1] for i in range(0, len(parts) - 1, 2)]
        if len(parts) % 2:
            nxt.append(parts[-1])
        parts = nxt
    return parts[0]


def _tc_kernel(sums2_ref, labels_ref, q_ref, bfm2_ref, out_ref,
               p_s, macc, tacc, cacc):
    i = pl.program_id(0)

    @pl.when(i == 0)
    def _():
        sums = sums2_ref[0] + sums2_ref[1]
        s2 = jnp.sum(sums * sums, axis=1, keepdims=True)
        centn = (sums * jax.lax.rsqrt(jnp.maximum(s2, 1e-24))).astype(
            jnp.bfloat16)
        bf = bfm2_ref[...] * jnp.bfloat16(-0.5)
        sim = jax.lax.dot_general(
            bf, centn, (((1,), (1,)), ((), ())),
            preferred_element_type=jnp.float32)
        col = jax.lax.broadcasted_iota(jnp.int32, (B, CPAD), 1)
        sim = jnp.where(col < C, sim, -3.0)
        p = jnp.argmax(sim, axis=1).astype(jnp.int32)
        p_s[...] = p[:, None]
        macc[...] = jnp.zeros_like(macc)
        tacc[...] = jnp.zeros_like(tacc)
        cacc[...] = jnp.zeros_like(cacc)

    @pl.when(i > 0)
    def _():
        simn2 = jax.lax.dot_general(
            bfm2_ref[...], q_ref[...], (((1,), (1,)), ((), ())),
            preferred_element_type=jnp.float32)
        t = simn2 + jnp.float32(2.0 + 1e-6)
        mae = (t * jax.lax.rsqrt(t)).astype(jnp.bfloat16)
        lab = labels_ref[0, 0, :]
        p16 = p_s[...].astype(jnp.int16)
        mask = p16 == lab[None, :]
        macc[...] += _lanegroup_sum(
            jnp.where(mask, mae, jnp.bfloat16(0.0))).astype(jnp.float32)
        tacc[...] += _lanegroup_sum(mae).astype(jnp.float32)
        cacc[...] += _lanegroup_sum(
            jnp.where(mask, jnp.bfloat16(1.0), jnp.bfloat16(0.0))
        ).astype(jnp.float32)

        @pl.when(i == NBLK)
        def _():
            m = jnp.sum(macc[...], axis=1)
            t_ = jnp.sum(tacc[...], axis=1)
            cnt = jnp.sum(cacc[...], axis=1)
            minent = jnp.mean(m / (cnt + 1e-6))
            inter = jnp.mean((t_ - m) / ((jnp.float32(K) - cnt) + 1e-6))
            out_ref[...] = jnp.broadcast_to(minent + 2.0 - inter, (1, 1))


def kernel(batch_feature, queue_emb_copy, info_label):
    lab2 = info_label.astype(jnp.int32).reshape(NW * NCH, CHUNK)
    labels3 = info_label.astype(jnp.int16).reshape(K // KBLK, 1, KBLK)
    q16 = queue_emb_copy.astype(jnp.bfloat16)
    bfm2 = (batch_feature * -2.0).astype(jnp.bfloat16)

    sc_seg = pl.kernel(
        _sc_segment_kernel,
        out_type=jax.ShapeDtypeStruct((NC, CPAD, D), jnp.float32),
        scratch_types=[
            pltpu.VMEM((NCH, CHUNK), jnp.int32),
            pltpu.VMEM((STAGE, D), jnp.float32),
            pltpu.VMEM((STAGE, D), jnp.float32),
            pltpu.VMEM((ZROWS, D), jnp.float32),
            pltpu.VMEM_SHARED((CPAD, D), jnp.float32),
            pltpu.SemaphoreType.DMA,
            pltpu.SemaphoreType.DMA,
        ],
        mesh=plsc.VectorSubcoreMesh(core_axis_name="c", subcore_axis_name="s",
                                    num_cores=NC, num_subcores=NS),
    )
    sums2 = sc_seg(queue_emb_copy, lab2)

    out = pl.pallas_call(
        _tc_kernel,
        grid=(NBLK + 1,),
        in_specs=[
            pl.BlockSpec((NC, CPAD, D), lambda i: (0, 0, 0)),
            pl.BlockSpec((1, 1, KBLK), lambda i: (jnp.maximum(i - 1, 0), 0, 0)),
            pl.BlockSpec((KBLK, D), lambda i: (jnp.maximum(i - 1, 0), 0)),
            pl.BlockSpec((B, D), lambda i: (0, 0)),
        ],
        out_specs=pl.BlockSpec((1, 1), lambda i: (0, 0)),
        out_shape=jax.ShapeDtypeStruct((1, 1), jnp.float32),
        scratch_shapes=[
            pltpu.VMEM((B, 1), jnp.int32),
            pltpu.VMEM((B, 128), jnp.float32),
            pltpu.VMEM((B, 128), jnp.float32),
            pltpu.VMEM((B, 128), jnp.float32),
        ],
    )(sums2, labels3, q16, bfm2)

    return out[0, 0]


# speedup vs baseline: 1.2141x; 1.0057x over previous
import jax
import jax.numpy as jnp
from jax import lax
from jax.experimental import pallas as pl
from jax.experimental.pallas import tpu as pltpu
from jax.experimental.pallas import tpu_sc as plsc

B = 1024
K = 65536
D = 128
C = 1000
CPAD = 1024
KBLK = 4096
NBLK = K // KBLK

NC = 2
NS = 16
NW = NC * NS
RPW = K // NW
CHUNK = 128
NCH = RPW // CHUNK


STAGE = 256
NSTG = RPW // STAGE
SCH = STAGE // CHUNK
ZROWS = CPAD // NS


def _sc_segment_kernel(q_hbm, lab2_hbm, sums_out,
                       lab_v, rows_a, rows_b, zero_v, sh_sums,
                       sem_a, sem_b):
    cid = lax.axis_index("c")
    sid = lax.axis_index("s")
    wid = cid * NS + sid

    for r in range(ZROWS):
        for g in range(D // 16):
            zero_v[r, pl.ds(g * 16, 16)] = jnp.zeros((16,), jnp.float32)
    pltpu.sync_copy(lab2_hbm.at[pl.ds(wid * NCH, NCH)], lab_v)
    pltpu.sync_copy(zero_v, sh_sums.at[pl.ds(sid * ZROWS, ZROWS)])

    plsc.subcore_barrier()

    bufs = (rows_a, rows_b)
    sems = (sem_a, sem_b)

    def _start(st):
        base = wid * RPW + st * STAGE
        pltpu.async_copy(q_hbm.at[pl.ds(base, STAGE)], bufs[st % 2],
                         sems[st % 2])

    def _wait(st):
        base = wid * RPW + st * STAGE
        pltpu.make_async_copy(q_hbm.at[pl.ds(base, STAGE)], bufs[st % 2],
                              sems[st % 2]).wait()

    _start(0)
    for st in range(NSTG):
        if st + 1 < NSTG:
            _start(st + 1)
        _wait(st)
        for c in range(SCH):
            pltpu.sync_copy(bufs[st % 2].at[pl.ds(c * CHUNK, CHUNK)],
                            sh_sums.at[lab_v.at[st * SCH + c]], add=True)

    plsc.subcore_barrier()

    @pl.when(sid == 0)
    def _():
        pltpu.sync_copy(sh_sums, sums_out.at[cid])


def _lanegroup_sum(x, width=128):
    parts = [x[:, g * width:(g + 1) * width] for g in range(x.shape[1] // width)]
    while len(parts) > 1:
        nxt = [parts[i] + parts[i + 1] for i in range(0, len(parts) - 1, 2)]
        if len(parts) % 2:
            nxt.append(parts[-1])
        parts = nxt
    return parts[0]


def _tc_kernel(sums2_ref, labels_ref, q_ref, bfm2_ref, out_ref,
               p_s, macc, tacc, cacc):
    i = pl.program_id(0)

    @pl.when(i == 0)
    def _():
        sums = sums2_ref[0] + sums2_ref[1]
        s2 = jnp.sum(sums * sums, axis=1, keepdims=True)
        centn = (sums * jax.lax.rsqrt(jnp.maximum(s2, 1e-24))).astype(
            jnp.bfloat16)
        bf = bfm2_ref[...] * jnp.bfloat16(-0.5)
        sim = jax.lax.dot_general(
            bf, centn, (((1,), (1,)), ((), ())),
            preferred_element_type=jnp.float32)
        col = jax.lax.broadcasted_iota(jnp.int32, (B, CPAD), 1)
        sim = jnp.where(col < C, sim, -3.0)
        p = jnp.argmax(sim, axis=1).astype(jnp.int32)
        p_s[...] = p[:, None]
        macc[...] = jnp.zeros_like(macc)
        tacc[...] = jnp.zeros_like(tacc)
        cacc[...] = jnp.zeros_like(cacc)

    @pl.when(i > 0)
    def _():
        simn2 = jax.lax.dot_general(
            bfm2_ref[...], q_ref[...], (((1,), (1,)), ((), ())),
            preferred_element_type=jnp.float32)
        t = simn2 + jnp.float32(2.0 + 1e-6)
        mae = (t * jax.lax.rsqrt(t)).astype(jnp.bfloat16)
        lab = labels_ref[0, 0, :]
        p16 = p_s[...].astype(jnp.int16)
        mask = p16 == lab[None, :]
        macc[...] += _lanegroup_sum(
            jnp.where(mask, mae, jnp.bfloat16(0.0))).astype(jnp.float32)
        tacc[...] += _lanegroup_sum(mae).astype(jnp.float32)
        cacc[...] += _lanegroup_sum(
            jnp.where(mask, jnp.bfloat16(1.0), jnp.bfloat16(0.0))
        ).astype(jnp.float32)

        @pl.when(i == NBLK)
        def _():
            m = jnp.sum(macc[...], axis=1)
            t_ = jnp.sum(tacc[...], axis=1)
            cnt = jnp.sum(cacc[...], axis=1)
            minent = jnp.mean(m / (cnt + 1e-6))
            inter = jnp.mean((t_ - m) / ((jnp.float32(K) - cnt) + 1e-6))
            out_ref[...] = jnp.broadcast_to(minent + 2.0 - inter, (1, 1))


def kernel(batch_feature, queue_emb_copy, info_label):
    lab2 = info_label.astype(jnp.int32).reshape(NW * NCH, CHUNK)
    labels3 = info_label.astype(jnp.int16).reshape(K // KBLK, 1, KBLK)
    q16 = queue_emb_copy.astype(jnp.bfloat16)
    bfm2 = (batch_feature * -2.0).astype(jnp.bfloat16)

    sc_seg = pl.kernel(
        _sc_segment_kernel,
        out_type=jax.ShapeDtypeStruct((NC, CPAD, D), jnp.float32),
        scratch_types=[
            pltpu.VMEM((NCH, CHUNK), jnp.int32),
            pltpu.VMEM((STAGE, D), jnp.float32),
            pltpu.VMEM((STAGE, D), jnp.float32),
            pltpu.VMEM((ZROWS, D), jnp.float32),
            pltpu.VMEM_SHARED((CPAD, D), jnp.float32),
            pltpu.SemaphoreType.DMA,
            pltpu.SemaphoreType.DMA,
        ],
        mesh=plsc.VectorSubcoreMesh(core_axis_name="c", subcore_axis_name="s",
                                    num_cores=NC, num_subcores=NS),
    )
    sums2 = sc_seg(queue_emb_copy, lab2)

    out = pl.pallas_call(
        _tc_kernel,
        grid=(NBLK + 1,),
        in_specs=[
            pl.BlockSpec((NC, CPAD, D), lambda i: (0, 0, 0)),
            pl.BlockSpec((1, 1, KBLK), lambda i: (jnp.maximum(i - 1, 0), 0, 0)),
            pl.BlockSpec((KBLK, D), lambda i: (jnp.maximum(i - 1, 0), 0)),
            pl.BlockSpec((B, D), lambda i: (0, 0)),
        ],
        out_specs=pl.BlockSpec((1, 1), lambda i: (0, 0)),
        out_shape=jax.ShapeDtypeStruct((1, 1), jnp.float32),
        scratch_shapes=[
            pltpu.VMEM((B, 1), jnp.int32),
            pltpu.VMEM((B, 128), jnp.float32),
            pltpu.VMEM((B, 128), jnp.float32),
            pltpu.VMEM((B, 128), jnp.float32),
        ],
    )(sums2, labels3, q16, bfm2)

    return out[0, 0]
